# Initial kernel scaffold; baseline (speedup 1.0000x reference)
import jax
import jax.numpy as jnp
from jax.experimental import pallas as pl


def kernel(pos, batch):
    raise NotImplementedError("write your pallas kernel here")


# speedup vs baseline: 8.5781x; 8.5781x over previous
import functools

import jax
import jax.numpy as jnp
from jax.experimental import pallas as pl
from jax.experimental.pallas import tpu as pltpu

_K = 32
_CUTOFF = 10.0
_N = 4096
_TILE_R = 256


def _topk_tile_kernel(xc, yc, zc, bc, xr, yr, zr, br, out_i, out_w, vmat):
    tile = pl.program_id(0)
    row_ids = (
        jax.lax.broadcasted_iota(jnp.int32, (_TILE_R, 1), 0) + tile * _TILE_R
    )
    col_ids = jax.lax.broadcasted_iota(jnp.int32, (1, _N), 1)

    dx = xc[...] - xr[...]
    dy = yc[...] - yr[...]
    dz = zc[...] - zr[...]
    sq = dx * dx + dy * dy + dz * dz
    safe = jnp.where(sq > 0.0, sq, 1.0)
    pdist = jnp.where(sq > 0.0, jnp.sqrt(safe), 0.0)

    mask = (row_ids == col_ids) | (bc[...] != br[...]) | (pdist > _CUTOFF)
    vmat[...] = jnp.where(mask, _CUTOFF, pdist)

    col_ids_b = jax.lax.broadcasted_iota(jnp.int32, (_TILE_R, _N), 1)
    k_ids = jax.lax.broadcasted_iota(jnp.int32, (_TILE_R, _K), 1)

    def body(k, carry):
        acc_w, acc_i = carry
        v = vmat[...]
        m = jnp.min(v, axis=1, keepdims=True)
        eq = v == m
        j = jnp.min(jnp.where(eq, col_ids_b, _N), axis=1, keepdims=True)
        acc_w = jnp.where(k_ids == k, m, acc_w)
        acc_i = jnp.where(k_ids == k, j, acc_i)
        vmat[...] = jnp.where(col_ids_b == j, jnp.inf, v)
        return acc_w, acc_i

    acc_w = jnp.zeros((_TILE_R, _K), jnp.float32)
    acc_i = jnp.zeros((_TILE_R, _K), jnp.int32)
    acc_w, acc_i = jax.lax.fori_loop(0, _K, body, (acc_w, acc_i))
    out_w[...] = acc_w
    out_i[...] = acc_i


@jax.jit
def kernel(pos, batch):
    n = pos.shape[0]
    pos = pos.astype(jnp.float32)
    batch = batch.astype(jnp.int32)

    xc = pos[:, 0:1]
    yc = pos[:, 1:2]
    zc = pos[:, 2:3]
    bc = batch[:, None]
    xr = pos[:, 0].reshape(1, n)
    yr = pos[:, 1].reshape(1, n)
    zr = pos[:, 2].reshape(1, n)
    br = batch.reshape(1, n)

    grid = n // _TILE_R
    col_spec = pl.BlockSpec((_TILE_R, 1), lambda i: (i, 0))
    row_spec = pl.BlockSpec((1, n), lambda i: (0, 0))

    out_i, out_w = pl.pallas_call(
        _topk_tile_kernel,
        grid=(grid,),
        in_specs=[col_spec, col_spec, col_spec, col_spec,
                  row_spec, row_spec, row_spec, row_spec],
        out_specs=[
            pl.BlockSpec((_TILE_R, _K), lambda i: (i, 0)),
            pl.BlockSpec((_TILE_R, _K), lambda i: (i, 0)),
        ],
        out_shape=[
            jax.ShapeDtypeStruct((n, _K), jnp.int32),
            jax.ShapeDtypeStruct((n, _K), jnp.float32),
        ],
        scratch_shapes=[pltpu.VMEM((_TILE_R, _N), jnp.float32)],
    )(xc, yc, zc, bc, xr, yr, zr, br)

    rows_e = jnp.broadcast_to(jnp.arange(n, dtype=jnp.int32)[:, None], (n, _K))
    edge_index = jnp.stack([out_i.reshape(-1), rows_e.reshape(-1)])
    edge_weight = out_w.reshape(-1)
    return edge_index, edge_weight


# speedup vs baseline: 60.3525x; 7.0356x over previous
import functools

import jax
import jax.numpy as jnp
from jax import lax
from jax.experimental import pallas as pl
from jax.experimental.pallas import tpu as pltpu
from jax.experimental.pallas import tpu_sc as plsc

_K = 32
_CUTOFF = 10.0
_N = 4096
_L = 16
_NW = 32
_RPW = _N // _NW
_BITS100 = 0x42C80000
_HUGE = 0x7F000000


def _bsearch16(bt_ref, keys):
    lo = jnp.zeros((_L,), jnp.int32)
    hi = jnp.full((_L,), _N, jnp.int32)
    for _ in range(12):
        mid = (lo + hi) >> 1
        vals = plsc.load_gather(bt_ref, [mid])
        cond = vals < keys
        lo = jnp.where(cond, mid + 1, lo)
        hi = jnp.where(cond, hi, mid)
    return lo


def _merge_chunk(key, val, t0k, t0v, t1k, t1v):
    ck, cv = plsc.sort_key_val(key, val)
    rck = lax.rev(ck, (0,))
    rcv = lax.rev(cv, (0,))
    c1 = rck < t1k
    l1k = jnp.where(c1, rck, t1k)
    l1v = jnp.where(c1, rcv, t1v)
    sl1k, sl1v = plsc.sort_key_val(l1k, l1v)
    rl1k = lax.rev(sl1k, (0,))
    rl1v = lax.rev(sl1v, (0,))
    c2 = rl1k < t0k
    l2k = jnp.where(c2, rl1k, t0k)
    l2v = jnp.where(c2, rl1v, t0v)
    h2k = jnp.where(c2, t0k, rl1k)
    h2v = jnp.where(c2, t0v, rl1v)
    nt0k, nt0v = plsc.sort_key_val(l2k, l2v)
    nt1k, nt1v = plsc.sort_key_val(h2k, h2v)
    return nt0k, nt0v, nt1k, nt1v


def _sc_topk():
    mesh = plsc.VectorSubcoreMesh(core_axis_name="c", subcore_axis_name="s")

    @functools.partial(
        pl.kernel,
        mesh=mesh,
        compiler_params=pltpu.CompilerParams(needs_layout_passes=False),
        out_type=[
            jax.ShapeDtypeStruct((_N * _K,), jnp.float32),
            jax.ShapeDtypeStruct((_N * _K,), jnp.int32),
        ],
        scratch_types=[
            pltpu.VMEM((_N + _L,), jnp.float32),
            pltpu.VMEM((_N + _L,), jnp.float32),
            pltpu.VMEM((_N + _L,), jnp.float32),
            pltpu.VMEM((_N + _L,), jnp.int32),
            pltpu.VMEM((_RPW + _L,), jnp.int32),
            pltpu.VMEM((_RPW + _L,), jnp.int32),
            pltpu.VMEM((_RPW * _K,), jnp.float32),
            pltpu.VMEM((_RPW * _K,), jnp.int32),
        ],
    )
    def kfn(xs_h, ys_h, zs_h, bt_h, osq_h, oidx_h,
            xs, ys, zs, bt, seg_s, seg_e, osq, oidx):
        wid = lax.axis_index("c") * 16 + lax.axis_index("s")
        base = wid * _RPW

        pltpu.sync_copy(xs_h, xs.at[pl.ds(0, _N)])
        pltpu.sync_copy(ys_h, ys.at[pl.ds(0, _N)])
        pltpu.sync_copy(zs_h, zs.at[pl.ds(0, _N)])
        pltpu.sync_copy(bt_h, bt.at[pl.ds(0, _N)])

        lane = lax.iota(jnp.int32, _L)
        for g in range(_RPW // _L):
            bb = bt[pl.ds(base + g * _L, _L)]
            seg_s[pl.ds(g * _L, _L)] = _bsearch16(bt, bb)
            seg_e[pl.ds(g * _L, _L)] = _bsearch16(bt, bb + 1)

        def row_body(r, _):
            i = base + r
            s_r = seg_s[pl.ds(r, _L)][0]
            e_r = seg_e[pl.ds(r, _L)][0]
            bx = jnp.full((_L,), xs[pl.ds(i, _L)][0], jnp.float32)
            by = jnp.full((_L,), ys[pl.ds(i, _L)][0], jnp.float32)
            bz = jnp.full((_L,), zs[pl.ds(i, _L)][0], jnp.float32)

            def chunk_keys(j0):
                jvec = lane + j0
                cx = xs[pl.ds(j0, _L)]
                cy = ys[pl.ds(j0, _L)]
                cz = zs[pl.ds(j0, _L)]
                dx = bx - cx
                dy = by - cy
                dz = bz - cz
                sq = dx * dx + dy * dy + dz * dz
                kbits = plsc.bitcast(sq, jnp.int32)
                masked = (
                    (jvec < s_r) | (jvec >= e_r) | (jvec == i)
                    | (sq >= _CUTOFF * _CUTOFF)
                )
                return jnp.where(masked, _BITS100 + jvec, kbits), jvec

            def maybe_merge(key, val, carry):
                t0k, t0v, t1k, t1v = carry
                kmin = jnp.min(key)
                t1max = jnp.max(t1k)
                return lax.cond(
                    kmin < t1max,
                    lambda: _merge_chunk(key, val, t0k, t0v, t1k, t1v),
                    lambda: (t0k, t0v, t1k, t1v),
                )

            t0k = jnp.full((_L,), _HUGE, jnp.int32)
            t1k = jnp.full((_L,), _HUGE, jnp.int32)
            t0v = jnp.zeros((_L,), jnp.int32)
            t1v = jnp.zeros((_L,), jnp.int32)
            carry = (t0k, t0v, t1k, t1v)

            c_lo = jnp.maximum(s_r >> 4, 4)
            c_hi = jnp.maximum((e_r + _L - 1) >> 4, 4)

            def seg_body(c, carry):
                key, val = chunk_keys(pl.multiple_of(c * _L, _L))
                return maybe_merge(key, val, carry)

            carry = lax.fori_loop(c_lo, c_hi, seg_body, carry)

            for c in range(4):
                key, val = chunk_keys(c * _L)
                carry = maybe_merge(key, val, carry)

            t0k, t0v, t1k, t1v = carry
            sq0 = jnp.where(
                t0k >= _BITS100, _CUTOFF * _CUTOFF,
                plsc.bitcast(t0k, jnp.float32))
            sq1 = jnp.where(
                t1k >= _BITS100, _CUTOFF * _CUTOFF,
                plsc.bitcast(t1k, jnp.float32))
            o = pl.multiple_of(r * _K, _K)
            osq[pl.ds(o, _L)] = sq0
            osq[pl.ds(o + _L, _L)] = sq1
            oidx[pl.ds(o, _L)] = t0v
            oidx[pl.ds(o + _L, _L)] = t1v
            return 0

        lax.fori_loop(0, _RPW, row_body, 0)

        pltpu.sync_copy(osq, osq_h.at[pl.ds(base * _K, _RPW * _K)])
        pltpu.sync_copy(oidx, oidx_h.at[pl.ds(base * _K, _RPW * _K)])

    return kfn


def _sqrt_kernel(sq_ref, w_ref):
    sq = sq_ref[...]
    safe = jnp.where(sq > 0.0, sq, 1.0)
    w_ref[...] = jnp.minimum(
        jnp.where(sq > 0.0, jnp.sqrt(safe), 0.0), _CUTOFF)


@jax.jit
def kernel(pos, batch):
    n = pos.shape[0]
    pos = pos.astype(jnp.float32)
    batch = batch.astype(jnp.int32)

    xs = pos[:, 0]
    ys = pos[:, 1]
    zs = pos[:, 2]

    osq, oidx = _sc_topk()(xs, ys, zs, batch)

    w = pl.pallas_call(
        _sqrt_kernel,
        out_shape=jax.ShapeDtypeStruct((n * _K // 128, 128), jnp.float32),
    )(osq.reshape(n * _K // 128, 128))

    rows_e = jnp.broadcast_to(
        jnp.arange(n, dtype=jnp.int32)[:, None], (n, _K))
    edge_index = jnp.stack([oidx, rows_e.reshape(-1)])
    edge_weight = w.reshape(-1)
    return edge_index, edge_weight


# speedup vs baseline: 121.8224x; 2.0185x over previous
import functools

import jax
import jax.numpy as jnp
from jax import lax
from jax.experimental import pallas as pl
from jax.experimental.pallas import tpu as pltpu
from jax.experimental.pallas import tpu_sc as plsc

_K = 32
_CUTOFF = 10.0
_N = 4096
_L = 16
_NW = 32
_RPW = _N // _NW
_BITS100 = 0x42C80000
_HUGE = 0x7F000000


def _bsearch16(bt_ref, keys):
    lo = jnp.zeros((_L,), jnp.int32)
    hi = jnp.full((_L,), _N, jnp.int32)
    for _ in range(12):
        mid = (lo + hi) >> 1
        vals = plsc.load_gather(bt_ref, [mid])
        cond = vals < keys
        lo = jnp.where(cond, mid + 1, lo)
        hi = jnp.where(cond, hi, mid)
    return lo


def _merge32(c1k, c1v, c2k, c2v):
    rk = lax.rev(c2k, (0,))
    rv = lax.rev(c2v, (0,))
    c = rk < c1k
    lk = jnp.where(c, rk, c1k)
    lv = jnp.where(c, rv, c1v)
    hk = jnp.where(c, c1k, rk)
    hv = jnp.where(c, c1v, rv)
    lk, lv = plsc.sort_key_val(lk, lv)
    hk, hv = plsc.sort_key_val(hk, hv)
    return lk, lv, hk, hv


def _tmerge(t, c0k, c0v, c1k, c1v):
    t0k, t0v, t1k, t1v = t
    r1k = lax.rev(c1k, (0,))
    r1v = lax.rev(c1v, (0,))
    r0k = lax.rev(c0k, (0,))
    r0v = lax.rev(c0v, (0,))
    c = r1k < t0k
    l0k = jnp.where(c, r1k, t0k)
    l0v = jnp.where(c, r1v, t0v)
    c = r0k < t1k
    l1k = jnp.where(c, r0k, t1k)
    l1v = jnp.where(c, r0v, t1v)
    c = l1k < l0k
    b0k = jnp.where(c, l1k, l0k)
    b0v = jnp.where(c, l1v, l0v)
    b1k = jnp.where(c, l0k, l1k)
    b1v = jnp.where(c, l0v, l1v)
    b0k, b0v = plsc.sort_key_val(b0k, b0v)
    b1k, b1v = plsc.sort_key_val(b1k, b1v)
    return b0k, b0v, b1k, b1v


def _sc_topk():
    mesh = plsc.VectorSubcoreMesh(core_axis_name="c", subcore_axis_name="s")

    @functools.partial(
        pl.kernel,
        mesh=mesh,
        compiler_params=pltpu.CompilerParams(needs_layout_passes=False),
        out_type=[
            jax.ShapeDtypeStruct((_N * _K,), jnp.float32),
            jax.ShapeDtypeStruct((_N * _K,), jnp.int32),
        ],
        scratch_types=[
            pltpu.VMEM((_N + _L,), jnp.float32),
            pltpu.VMEM((_N + _L,), jnp.float32),
            pltpu.VMEM((_N + _L,), jnp.float32),
            pltpu.VMEM((_N + _L,), jnp.int32),
            pltpu.VMEM((_RPW + _L,), jnp.int32),
            pltpu.VMEM((_RPW + _L,), jnp.int32),
            pltpu.VMEM((_RPW * _K,), jnp.float32),
            pltpu.VMEM((_RPW * _K,), jnp.int32),
        ],
    )
    def kfn(xs_h, ys_h, zs_h, bt_h, osq_h, oidx_h,
            xs, ys, zs, bt, seg_s, seg_e, osq, oidx):
        wid = lax.axis_index("c") * 16 + lax.axis_index("s")
        base = wid * _RPW

        pltpu.sync_copy(xs_h, xs.at[pl.ds(0, _N)])
        pltpu.sync_copy(ys_h, ys.at[pl.ds(0, _N)])
        pltpu.sync_copy(zs_h, zs.at[pl.ds(0, _N)])
        pltpu.sync_copy(bt_h, bt.at[pl.ds(0, _N)])

        lane = lax.iota(jnp.int32, _L)
        for g in range(_RPW // _L):
            bb = bt[pl.ds(base + g * _L, _L)]
            seg_s[pl.ds(g * _L, _L)] = _bsearch16(bt, bb)
            seg_e[pl.ds(g * _L, _L)] = _bsearch16(bt, bb + 1)

        def pair_body(p, _):
            r0 = 2 * p
            i0 = base + r0
            i1 = i0 + 1
            sv = seg_s[pl.ds(r0, _L)]
            ev = seg_e[pl.ds(r0, _L)]
            s0, s1 = sv[0], sv[1]
            e0, e1 = ev[0], ev[1]
            vx = xs[pl.ds(i0, _L)]
            vy = ys[pl.ds(i0, _L)]
            vz = zs[pl.ds(i0, _L)]
            rowa = (i0, s0, e0, vx[0], vy[0], vz[0])
            rowb = (i1, s1, e1, vx[1], vy[1], vz[1])

            def keys_for(row, cx, cy, cz, jvec):
                i, s_r, e_r, bx, by, bz = row
                dx = bx - cx
                dy = by - cy
                dz = bz - cz
                sq = dx * dx + dy * dy + dz * dz
                kbits = plsc.bitcast(sq, jnp.int32)
                masked = (
                    (jvec < s_r) | (jvec >= e_r) | (jvec == i)
                    | (sq >= _CUTOFF * _CUTOFF)
                )
                return jnp.where(masked, _BITS100 + jvec, kbits), jvec

            def super_chunk(j0, ta, tb):
                j1 = j0 + _L
                cx1 = xs[pl.ds(j0, _L)]
                cy1 = ys[pl.ds(j0, _L)]
                cz1 = zs[pl.ds(j0, _L)]
                cx2 = xs[pl.ds(j1, _L)]
                cy2 = ys[pl.ds(j1, _L)]
                cz2 = zs[pl.ds(j1, _L)]
                jv1 = lane + j0
                jv2 = lane + j1
                outs = []
                for row, t in ((rowa, ta), (rowb, tb)):
                    k1, v1 = keys_for(row, cx1, cy1, cz1, jv1)
                    k2, v2 = keys_for(row, cx2, cy2, cz2, jv2)
                    k1, v1 = plsc.sort_key_val(k1, v1)
                    k2, v2 = plsc.sort_key_val(k2, v2)
                    c0k, c0v, c1k, c1v = _merge32(k1, v1, k2, v2)
                    outs.append(_tmerge(t, c0k, c0v, c1k, c1v))
                return tuple(outs[0]), tuple(outs[1])

            init = (
                jnp.full((_L,), _HUGE, jnp.int32),
                jnp.zeros((_L,), jnp.int32),
                jnp.full((_L,), _HUGE, jnp.int32),
                jnp.zeros((_L,), jnp.int32),
            )

            s_pair = jnp.minimum(s0, s1)
            e_pair = jnp.maximum(e0, e1)
            c_lo = jnp.maximum(s_pair >> 5, 2)
            c_hi = jnp.maximum((e_pair + 31) >> 5, 2)

            def seg_body(c, carry):
                ta, tb = carry
                return super_chunk(pl.multiple_of(c * 2 * _L, 2 * _L), ta, tb)

            ta, tb = lax.fori_loop(c_lo, c_hi, seg_body, (init, init))

            def do_prefix(ta, tb):
                ta, tb = super_chunk(0, ta, tb)
                return super_chunk(2 * _L, ta, tb)

            need = (
                (s_pair < 4 * _L) | (ta[2][_L - 1] >= _BITS100)
                | (tb[2][_L - 1] >= _BITS100)
            )
            ta, tb = lax.cond(need, do_prefix, lambda ta, tb: (ta, tb), ta, tb)

            for r, (t0k, t0v, t1k, t1v) in ((r0, ta), (r0 + 1, tb)):
                sq0 = jnp.where(
                    t0k >= _BITS100, _CUTOFF * _CUTOFF,
                    plsc.bitcast(t0k, jnp.float32))
                sq1 = jnp.where(
                    t1k >= _BITS100, _CUTOFF * _CUTOFF,
                    plsc.bitcast(t1k, jnp.float32))
                o = pl.multiple_of(r * _K, _K)
                osq[pl.ds(o, _L)] = sq0
                osq[pl.ds(o + _L, _L)] = sq1
                oidx[pl.ds(o, _L)] = t0v
                oidx[pl.ds(o + _L, _L)] = t1v
            return 0

        lax.fori_loop(0, _RPW // 2, pair_body, 0)

        pltpu.sync_copy(osq, osq_h.at[pl.ds(base * _K, _RPW * _K)])
        pltpu.sync_copy(oidx, oidx_h.at[pl.ds(base * _K, _RPW * _K)])

    return kfn


def _sqrt_kernel(sq_ref, w_ref):
    sq = sq_ref[...]
    safe = jnp.where(sq > 0.0, sq, 1.0)
    w_ref[...] = jnp.minimum(
        jnp.where(sq > 0.0, jnp.sqrt(safe), 0.0), _CUTOFF)


@jax.jit
def kernel(pos, batch):
    n = pos.shape[0]
    pos = pos.astype(jnp.float32)
    batch = batch.astype(jnp.int32)

    xs = pos[:, 0]
    ys = pos[:, 1]
    zs = pos[:, 2]

    osq, oidx = _sc_topk()(xs, ys, zs, batch)

    w = pl.pallas_call(
        _sqrt_kernel,
        out_shape=jax.ShapeDtypeStruct((n * _K // 128, 128), jnp.float32),
    )(osq.reshape(n * _K // 128, 128))

    rows_e = jnp.broadcast_to(
        jnp.arange(n, dtype=jnp.int32)[:, None], (n, _K))
    edge_index = jnp.stack([oidx, rows_e.reshape(-1)])
    edge_weight = w.reshape(-1)
    return edge_index, edge_weight


# speedup vs baseline: 124.6390x; 1.0231x over previous
import functools

import jax
import jax.numpy as jnp
from jax import lax
from jax.experimental import pallas as pl
from jax.experimental.pallas import tpu as pltpu
from jax.experimental.pallas import tpu_sc as plsc

_K = 32
_CUTOFF = 10.0
_N = 4096
_L = 16
_NW = 32
_RPW = _N // _NW
_BITS100 = 0x42C80000
_HUGE = 0x7F000000


def _bsearch16(bt_ref, keys):
    lo = jnp.zeros((_L,), jnp.int32)
    hi = jnp.full((_L,), _N, jnp.int32)
    for _ in range(12):
        mid = (lo + hi) >> 1
        vals = plsc.load_gather(bt_ref, [mid])
        cond = vals < keys
        lo = jnp.where(cond, mid + 1, lo)
        hi = jnp.where(cond, hi, mid)
    return lo


def _merge32(c1k, c1v, c2k, c2v):
    c = c2k < c1k
    lk = jnp.where(c, c2k, c1k)
    lv = jnp.where(c, c2v, c1v)
    hk = jnp.where(c, c1k, c2k)
    hv = jnp.where(c, c1v, c2v)
    lk, lv = plsc.sort_key_val(lk, lv, descending=True)
    hk, hv = plsc.sort_key_val(hk, hv, descending=True)
    return lk, lv, hk, hv


def _tmerge(t, c0k, c0v, c1k, c1v):
    t0k, t0v, t1k, t1v = t
    c = c1k < t0k
    l0k = jnp.where(c, c1k, t0k)
    l0v = jnp.where(c, c1v, t0v)
    c = c0k < t1k
    l1k = jnp.where(c, c0k, t1k)
    l1v = jnp.where(c, c0v, t1v)
    c = l1k < l0k
    b0k = jnp.where(c, l1k, l0k)
    b0v = jnp.where(c, l1v, l0v)
    b1k = jnp.where(c, l0k, l1k)
    b1v = jnp.where(c, l0v, l1v)
    b0k, b0v = plsc.sort_key_val(b0k, b0v)
    b1k, b1v = plsc.sort_key_val(b1k, b1v)
    return b0k, b0v, b1k, b1v


def _sqrt16(s):
    sc = jnp.maximum(s, 1e-30)
    b = plsc.bitcast(sc, jnp.int32)
    y = plsc.bitcast(0x5F3759DF - (b >> 1), jnp.float32)
    h = 0.5 * sc
    y = y * (1.5 - h * y * y)
    y = y * (1.5 - h * y * y)
    w = sc * y
    w = 0.5 * (w + sc / w)
    return jnp.where(s > 0.0, w, 0.0)


def _sc_topk():
    mesh = plsc.VectorSubcoreMesh(core_axis_name="c", subcore_axis_name="s")

    @functools.partial(
        pl.kernel,
        mesh=mesh,
        compiler_params=pltpu.CompilerParams(needs_layout_passes=False),
        out_type=[
            jax.ShapeDtypeStruct((2 * _N * _K,), jnp.int32),
            jax.ShapeDtypeStruct((_N * _K,), jnp.float32),
        ],
        scratch_types=[
            pltpu.VMEM((_N + _L,), jnp.float32),
            pltpu.VMEM((_N + _L,), jnp.float32),
            pltpu.VMEM((_N + _L,), jnp.float32),
            pltpu.VMEM((_N + _L,), jnp.int32),
            pltpu.VMEM((_RPW + _L,), jnp.int32),
            pltpu.VMEM((_RPW + _L,), jnp.int32),
            pltpu.VMEM((_RPW * _K,), jnp.float32),
            pltpu.VMEM((_RPW * _K,), jnp.int32),
            pltpu.VMEM((_RPW * _K,), jnp.int32),
        ],
    )
    def kfn(xs_h, ys_h, zs_h, bt_h, ei_h, ew_h,
            xs, ys, zs, bt, seg_s, seg_e, ow, oidx, orow):
        wid = lax.axis_index("c") * 16 + lax.axis_index("s")
        base = wid * _RPW

        pltpu.sync_copy(xs_h, xs.at[pl.ds(0, _N)])
        pltpu.sync_copy(ys_h, ys.at[pl.ds(0, _N)])
        pltpu.sync_copy(zs_h, zs.at[pl.ds(0, _N)])
        pltpu.sync_copy(bt_h, bt.at[pl.ds(0, _N)])

        lane = lax.iota(jnp.int32, _L)
        for g in range(_RPW // _L):
            bb = bt[pl.ds(base + g * _L, _L)]
            seg_s[pl.ds(g * _L, _L)] = _bsearch16(bt, bb)
            seg_e[pl.ds(g * _L, _L)] = _bsearch16(bt, bb + 1)

        def pair_body(p, _):
            r0 = 2 * p
            i0 = base + r0
            i1 = i0 + 1
            sv = seg_s[pl.ds(r0, _L)]
            ev = seg_e[pl.ds(r0, _L)]
            s0, s1 = sv[0], sv[1]
            e0, e1 = ev[0], ev[1]
            vx = xs[pl.ds(i0, _L)]
            vy = ys[pl.ds(i0, _L)]
            vz = zs[pl.ds(i0, _L)]
            rowa = (i0, s0, e0, vx[0], vy[0], vz[0])
            rowb = (i1, s1, e1, vx[1], vy[1], vz[1])

            def keys_for(row, cx, cy, cz, jvec):
                i, s_r, e_r, bx, by, bz = row
                dx = bx - cx
                dy = by - cy
                dz = bz - cz
                sq = dx * dx + dy * dy + dz * dz
                kbits = plsc.bitcast(sq, jnp.int32)
                masked = (
                    (jvec < s_r) | (jvec >= e_r) | (jvec == i)
                    | (sq >= _CUTOFF * _CUTOFF)
                )
                return jnp.where(masked, _BITS100 + jvec, kbits), jvec

            def super_chunk(j0, ta, tb):
                j1 = j0 + _L
                cx1 = xs[pl.ds(j0, _L)]
                cy1 = ys[pl.ds(j0, _L)]
                cz1 = zs[pl.ds(j0, _L)]
                cx2 = xs[pl.ds(j1, _L)]
                cy2 = ys[pl.ds(j1, _L)]
                cz2 = zs[pl.ds(j1, _L)]
                jv1 = lane + j0
                jv2 = lane + j1
                outs = []
                for row, t in ((rowa, ta), (rowb, tb)):
                    k1, v1 = keys_for(row, cx1, cy1, cz1, jv1)
                    k2, v2 = keys_for(row, cx2, cy2, cz2, jv2)
                    k1, v1 = plsc.sort_key_val(k1, v1)
                    k2, v2 = plsc.sort_key_val(k2, v2, descending=True)
                    c0k, c0v, c1k, c1v = _merge32(k1, v1, k2, v2)
                    outs.append(_tmerge(t, c0k, c0v, c1k, c1v))
                return tuple(outs[0]), tuple(outs[1])

            init = (
                jnp.full((_L,), _HUGE, jnp.int32),
                jnp.zeros((_L,), jnp.int32),
                jnp.full((_L,), _HUGE, jnp.int32),
                jnp.zeros((_L,), jnp.int32),
            )

            s_pair = jnp.minimum(s0, s1)
            e_pair = jnp.maximum(e0, e1)
            c_lo = jnp.maximum(s_pair >> 5, 2)
            c_hi = jnp.maximum((e_pair + 31) >> 5, 2)

            def seg_body(c, carry):
                ta, tb = carry
                return super_chunk(pl.multiple_of(c * 2 * _L, 2 * _L), ta, tb)

            ta, tb = lax.fori_loop(c_lo, c_hi, seg_body, (init, init))

            def do_prefix(ta, tb):
                ta, tb = super_chunk(0, ta, tb)
                return super_chunk(2 * _L, ta, tb)

            need = (
                (s_pair < 4 * _L) | (ta[2][_L - 1] >= _BITS100)
                | (tb[2][_L - 1] >= _BITS100)
            )
            ta, tb = lax.cond(need, do_prefix, lambda ta, tb: (ta, tb), ta, tb)

            for r, i, (t0k, t0v, t1k, t1v) in ((r0, i0, ta), (r0 + 1, i1, tb)):
                w0 = jnp.where(
                    t0k >= _BITS100, _CUTOFF,
                    _sqrt16(plsc.bitcast(t0k, jnp.float32)))
                w1 = jnp.where(
                    t1k >= _BITS100, _CUTOFF,
                    _sqrt16(plsc.bitcast(t1k, jnp.float32)))
                o = pl.multiple_of(r * _K, _K)
                ow[pl.ds(o, _L)] = w0
                ow[pl.ds(o + _L, _L)] = w1
                oidx[pl.ds(o, _L)] = t0v
                oidx[pl.ds(o + _L, _L)] = t1v
                ivec = jnp.full((_L,), i, jnp.int32)
                orow[pl.ds(o, _L)] = ivec
                orow[pl.ds(o + _L, _L)] = ivec
            return 0

        lax.fori_loop(0, _RPW // 2, pair_body, 0)

        pltpu.sync_copy(oidx, ei_h.at[pl.ds(base * _K, _RPW * _K)])
        pltpu.sync_copy(orow, ei_h.at[pl.ds(_N * _K + base * _K, _RPW * _K)])
        pltpu.sync_copy(ow, ew_h.at[pl.ds(base * _K, _RPW * _K)])

    return kfn


@jax.jit
def kernel(pos, batch):
    n = pos.shape[0]
    pos = pos.astype(jnp.float32)
    batch = batch.astype(jnp.int32)

    ei_flat, edge_weight = _sc_topk()(pos[:, 0], pos[:, 1], pos[:, 2], batch)
    return ei_flat.reshape(2, n * _K), edge_weight
